# Initial kernel scaffold; baseline (speedup 1.0000x reference)
import jax
import jax.numpy as jnp
from jax.experimental import pallas as pl


def kernel(x, edge_index, adj_values, W):
    raise NotImplementedError("write your pallas kernel here")


# speedup vs baseline: 4.4767x; 4.4767x over previous
import functools

import jax
import jax.numpy as jnp
from jax import lax
from jax.experimental import pallas as pl
from jax.experimental.pallas import tpu as pltpu
from jax.experimental.pallas import tpu_sc as plsc

_NC = 2
_NS = 16
_C = 80


def _mm_body(x_ref, w_ref, o_ref):
    o_ref[...] = jnp.dot(x_ref[...], w_ref[...],
                         preferred_element_type=jnp.float32)


def _merge_body(p_ref, o_ref):
    o_ref[...] = p_ref[0] + p_ref[1]


@functools.partial(jax.jit, static_argnums=(4, 5, 6))
def _agg(support, row, col, val, N, E, D):
    ept = E // (_NC * _NS)
    nchunk = ept // _C
    rpt = (N // _NS) // 8 * 8
    rem = N - _NS * rpt
    mesh = plsc.VectorSubcoreMesh(core_axis_name="c", subcore_axis_name="s")

    def body(sup, rowh, colh, valh, zeroh, out,
             colb, rowb, valb, gbuf, acc, sem):
        cid = lax.axis_index("c")
        sid = lax.axis_index("s")
        pltpu.sync_copy(zeroh.at[pl.ds(sid * rpt, rpt)],
                        acc.at[pl.ds(sid * rpt, rpt)])
        if rem:
            @pl.when(sid == 0)
            def _():
                pltpu.sync_copy(zeroh.at[pl.ds(_NS * rpt, rem)],
                                acc.at[pl.ds(_NS * rpt, rem)])
        plsc.subcore_barrier()
        base0 = (cid * _NS + sid) * ept

        def chunk(k, carry):
            base = base0 + k * _C
            pltpu.sync_copy(colh.at[pl.ds(base, _C)], colb)
            pltpu.sync_copy(valh.at[pl.ds(base, _C)], valb)
            pltpu.sync_copy(rowh.at[pl.ds(base, _C)], rowb)
            pltpu.async_copy(sup.at[colb], gbuf, sem).wait()
            for g in range(_C // 16):
                vv = valb[pl.ds(g * 16, 16)]
                for lane in range(16):
                    e = g * 16 + lane
                    vb = lax.gather(
                        vv, jnp.full((16, 1), lane, jnp.int32),
                        lax.GatherDimensionNumbers(
                            offset_dims=(), collapsed_slice_dims=(0,),
                            start_index_map=(0,)),
                        (1,),
                        mode=lax.GatherScatterMode.PROMISE_IN_BOUNDS)
                    for j in range(D // 16):
                        sl = pl.ds(j * 16, 16)
                        gbuf[e, sl] = gbuf[e, sl] * vb
            pltpu.sync_copy(gbuf, acc.at[rowb], add=True)
            return carry

        lax.fori_loop(0, nchunk, chunk, 0)
        plsc.subcore_barrier()
        pltpu.sync_copy(acc.at[pl.ds(sid * rpt, rpt)],
                        out.at[cid, pl.ds(sid * rpt, rpt)])
        if rem:
            @pl.when(sid == 0)
            def _():
                pltpu.sync_copy(acc.at[pl.ds(_NS * rpt, rem)],
                                out.at[cid, pl.ds(_NS * rpt, rem)])

    zeros = jnp.zeros((N, D), jnp.float32)
    agg = pl.kernel(
        body,
        out_type=jax.ShapeDtypeStruct((_NC, N, D), jnp.float32),
        mesh=mesh,
        scratch_types=[
            pltpu.VMEM((_C,), jnp.int32),
            pltpu.VMEM((_C,), jnp.int32),
            pltpu.VMEM((_C,), jnp.float32),
            pltpu.VMEM((_C, D), jnp.float32),
            pltpu.VMEM_SHARED((N, D), jnp.float32),
            pltpu.SemaphoreType.DMA,
        ],
    )
    return agg(support, row, col, val, zeros)


def kernel(x, edge_index, adj_values, W):
    N, _ = x.shape
    D = W.shape[1]
    E = adj_values.shape[0]
    rb = N // 5

    support = pl.pallas_call(
        _mm_body,
        grid=(5,),
        in_specs=[
            pl.BlockSpec((rb, x.shape[1]), lambda i: (i, 0)),
            pl.BlockSpec(W.shape, lambda i: (0, 0)),
        ],
        out_specs=pl.BlockSpec((rb, D), lambda i: (i, 0)),
        out_shape=jax.ShapeDtypeStruct((N, D), jnp.float32),
    )(x, W)

    row = edge_index[0]
    col = edge_index[1]
    partial = _agg(support, row, col, adj_values, N, E, D)

    out = pl.pallas_call(
        _merge_body,
        grid=(5,),
        in_specs=[pl.BlockSpec((_NC, rb, D), lambda i: (0, i, 0))],
        out_specs=pl.BlockSpec((rb, D), lambda i: (i, 0)),
        out_shape=jax.ShapeDtypeStruct((N, D), jnp.float32),
    )(partial)
    return out
